# Initial kernel scaffold; baseline (speedup 1.0000x reference)
import jax
import jax.numpy as jnp
from jax.experimental import pallas as pl


def kernel(token_embs, dep_embs, one_hot_embs, roots, token_mask, deplinks, path_idx, path_batch, W_iou0, U_iou0, b_iou0, W_f0, U_f0, b_f0, W_iou1, U_iou1, b_iou1, W_f1, U_f1, b_f1):
    raise NotImplementedError("write your pallas kernel here")


# speedup vs baseline: 3.6041x; 3.6041x over previous
import functools

import jax
import jax.numpy as jnp
from jax.experimental import pallas as pl
from jax.experimental.pallas import tpu as pltpu

B, S, P, L = 16, 256, 8192, 16
D_TOK, D_OH, D_DEP = 256, 50, 50
D = D_TOK + D_OH + D_DEP
DP = 384
H = 128
PB = 256


def _scan_body(x_ref, w0_ref, u0_ref, b0_ref, w1_ref, u1_ref, b1_ref, out_ref):
    f32 = jnp.float32
    bf16 = jnp.bfloat16
    h_up = jnp.zeros((PB, H), f32)
    c_up = jnp.zeros((PB, H), f32)
    h_dn = jnp.zeros((PB, H), f32)
    c_dn = jnp.zeros((PB, H), f32)
    h_dn_first = jnp.zeros((PB, H), f32)
    b0 = b0_ref[0, :]
    b1 = b1_ref[0, :]

    def cell(x_t, h, c, w_ref, u_ref, b):
        z = jnp.dot(x_t, w_ref[...], preferred_element_type=f32)
        z += jnp.dot(h.astype(bf16), u_ref[...], preferred_element_type=f32)
        z += b
        i = jax.nn.sigmoid(z[:, 0 * H:1 * H])
        o = jax.nn.sigmoid(z[:, 1 * H:2 * H])
        u = jnp.tanh(z[:, 2 * H:3 * H])
        f = jax.nn.sigmoid(z[:, 3 * H:4 * H])
        c_new = i * u + f * c
        h_new = o * jnp.tanh(c_new)
        return h_new, c_new

    for t in range(L):
        h_up, c_up = cell(x_ref[t], h_up, c_up, w0_ref, u0_ref, b0)
        h_dn, c_dn = cell(x_ref[L - 1 - t], h_dn, c_dn, w1_ref, u1_ref, b1)
        if t == 0:
            h_dn_first = h_dn

    out_ref[:, 0 * H:1 * H] = h_up
    out_ref[:, 1 * H:2 * H] = h_dn
    out_ref[:, 2 * H:3 * H] = h_dn_first


def _lstm_scan(x, w0, u0, b0, w1, u1, b1, interpret=False):
    grid = (P // PB,)
    return pl.pallas_call(
        _scan_body,
        grid=grid,
        in_specs=[
            pl.BlockSpec((L, PB, DP), lambda i: (0, i, 0)),
            pl.BlockSpec((DP, 4 * H), lambda i: (0, 0)),
            pl.BlockSpec((H, 4 * H), lambda i: (0, 0)),
            pl.BlockSpec((1, 4 * H), lambda i: (0, 0)),
            pl.BlockSpec((DP, 4 * H), lambda i: (0, 0)),
            pl.BlockSpec((H, 4 * H), lambda i: (0, 0)),
            pl.BlockSpec((1, 4 * H), lambda i: (0, 0)),
        ],
        out_specs=pl.BlockSpec((PB, 3 * H), lambda i: (i, 0)),
        out_shape=jax.ShapeDtypeStruct((P, 3 * H), jnp.float32),
        interpret=interpret,
    )(x, w0, u0, b0, w1, u1, b1)


def kernel(token_embs, dep_embs, one_hot_embs, roots, token_mask, deplinks,
           path_idx, path_batch,
           W_iou0, U_iou0, b_iou0, W_f0, U_f0, b_f0,
           W_iou1, U_iou1, b_iou1, W_f1, U_f1, b_f1):
    f32 = jnp.float32
    bf16 = jnp.bfloat16
    node = jnp.concatenate((token_embs, one_hot_embs, dep_embs), axis=-1)
    node = node * token_mask[..., None].astype(f32)
    table = jnp.pad(node.reshape(B * S, D), ((0, 0), (0, DP - D))).astype(bf16)

    def wcat(Wiou, Wf):
        return jnp.pad(jnp.concatenate((Wiou, Wf), axis=1),
                       ((0, DP - D), (0, 0))).astype(bf16)

    w0 = wcat(W_iou0, W_f0)
    w1 = wcat(W_iou1, W_f1)
    u0 = jnp.concatenate((U_iou0, U_f0), axis=1).astype(bf16)
    u1 = jnp.concatenate((U_iou1, U_f1), axis=1).astype(bf16)
    b0 = jnp.concatenate((b_iou0, b_f0))[None, :].astype(f32)
    b1 = jnp.concatenate((b_iou1, b_f1))[None, :].astype(f32)

    flat_idx = (path_batch[None, :] * S + path_idx.T).astype(jnp.int32)
    x = table[flat_idx.reshape(-1)].reshape(L, P, DP)

    return _lstm_scan(x, w0, u0, b0, w1, u1, b1)


# speedup vs baseline: 6.6491x; 1.8449x over previous
import functools

import jax
import jax.numpy as jnp
from jax import lax
from jax.experimental import pallas as pl
from jax.experimental.pallas import tpu as pltpu
from jax.experimental.pallas import tpu_sc as plsc

B, S, P, L = 16, 256, 8192, 16
D_TOK, D_OH, D_DEP = 256, 50, 50
D = D_TOK + D_OH + D_DEP
DP = 384
DW = DP
H = 128
PB = 256

NC, NS = 2, 16
NW = NC * NS
N_ROWS = L * P
RW = N_ROWS // NW
KCH = 128


def _gather_body(idx_hbm, table_hbm, out_hbm, idx_v, rows_v, sem):
    wid = lax.axis_index("s") * NC + lax.axis_index("c")
    base = wid * RW

    def chunk(j, carry):
        off = base + j * KCH
        pltpu.sync_copy(idx_hbm.at[pl.ds(off, KCH)], idx_v)
        pltpu.async_copy(table_hbm.at[idx_v], rows_v, sem).wait()
        pltpu.sync_copy(rows_v, out_hbm.at[pl.ds(off, KCH)])
        return carry

    lax.fori_loop(0, RW // KCH, chunk, 0)


def _sc_gather(flat_idx, table_w):
    mesh = plsc.VectorSubcoreMesh(core_axis_name="c", subcore_axis_name="s")
    return pl.kernel(
        _gather_body,
        out_type=jax.ShapeDtypeStruct((N_ROWS, DW), jnp.float32),
        mesh=mesh,
        scratch_types=[
            pltpu.VMEM((KCH,), jnp.int32),
            pltpu.VMEM((KCH, DW), jnp.float32),
            pltpu.SemaphoreType.DMA,
        ],
    )(flat_idx, table_w)


def _scan_body(x_ref, w0_ref, u0_ref, b0_ref, w1_ref, u1_ref, b1_ref, out_ref):
    f32 = jnp.float32
    bf16 = jnp.bfloat16
    h_up = jnp.zeros((PB, H), f32)
    c_up = jnp.zeros((PB, H), f32)
    h_dn = jnp.zeros((PB, H), f32)
    c_dn = jnp.zeros((PB, H), f32)
    h_dn_first = jnp.zeros((PB, H), f32)
    b0 = b0_ref[0, :]
    b1 = b1_ref[0, :]

    def cell(x_t, h, c, w_ref, u_ref, b):
        z = jnp.dot(x_t.astype(bf16), w_ref[...], preferred_element_type=f32)
        z += jnp.dot(h.astype(bf16), u_ref[...], preferred_element_type=f32)
        z += b
        i = jax.nn.sigmoid(z[:, 0 * H:1 * H])
        o = jax.nn.sigmoid(z[:, 1 * H:2 * H])
        u = jnp.tanh(z[:, 2 * H:3 * H])
        f = jax.nn.sigmoid(z[:, 3 * H:4 * H])
        c_new = i * u + f * c
        h_new = o * jnp.tanh(c_new)
        return h_new, c_new

    for t in range(L):
        h_up, c_up = cell(x_ref[t], h_up, c_up, w0_ref, u0_ref, b0)
        h_dn, c_dn = cell(x_ref[L - 1 - t], h_dn, c_dn, w1_ref, u1_ref, b1)
        if t == 0:
            h_dn_first = h_dn

    out_ref[:, 0 * H:1 * H] = h_up
    out_ref[:, 1 * H:2 * H] = h_dn
    out_ref[:, 2 * H:3 * H] = h_dn_first


def _lstm_scan(x, w0, u0, b0, w1, u1, b1, interpret=False):
    grid = (P // PB,)
    return pl.pallas_call(
        _scan_body,
        grid=grid,
        in_specs=[
            pl.BlockSpec((L, PB, DP), lambda i: (0, i, 0)),
            pl.BlockSpec((DP, 4 * H), lambda i: (0, 0)),
            pl.BlockSpec((H, 4 * H), lambda i: (0, 0)),
            pl.BlockSpec((1, 4 * H), lambda i: (0, 0)),
            pl.BlockSpec((DP, 4 * H), lambda i: (0, 0)),
            pl.BlockSpec((H, 4 * H), lambda i: (0, 0)),
            pl.BlockSpec((1, 4 * H), lambda i: (0, 0)),
        ],
        out_specs=pl.BlockSpec((PB, 3 * H), lambda i: (i, 0)),
        out_shape=jax.ShapeDtypeStruct((P, 3 * H), jnp.float32),
        interpret=interpret,
    )(x, w0, u0, b0, w1, u1, b1)


def kernel(token_embs, dep_embs, one_hot_embs, roots, token_mask, deplinks,
           path_idx, path_batch,
           W_iou0, U_iou0, b_iou0, W_f0, U_f0, b_f0,
           W_iou1, U_iou1, b_iou1, W_f1, U_f1, b_f1):
    f32 = jnp.float32
    bf16 = jnp.bfloat16
    node = jnp.concatenate((token_embs, one_hot_embs, dep_embs), axis=-1)
    node = node * token_mask[..., None].astype(f32)
    table = jnp.pad(node.reshape(B * S, D), ((0, 0), (0, DP - D)))

    def wcat(Wiou, Wf):
        return jnp.pad(jnp.concatenate((Wiou, Wf), axis=1),
                       ((0, DP - D), (0, 0))).astype(bf16)

    w0 = wcat(W_iou0, W_f0)
    w1 = wcat(W_iou1, W_f1)
    u0 = jnp.concatenate((U_iou0, U_f0), axis=1).astype(bf16)
    u1 = jnp.concatenate((U_iou1, U_f1), axis=1).astype(bf16)
    b0 = jnp.concatenate((b_iou0, b_f0))[None, :].astype(f32)
    b1 = jnp.concatenate((b_iou1, b_f1))[None, :].astype(f32)

    flat_idx = (path_batch[None, :] * S + path_idx.T).astype(jnp.int32)
    x = _sc_gather(flat_idx.reshape(-1), table).reshape(L, P, DP)

    return _lstm_scan(x, w0, u0, b0, w1, u1, b1)


# speedup vs baseline: 7.0913x; 1.0665x over previous
import functools

import jax
import jax.numpy as jnp
from jax import lax
from jax.experimental import pallas as pl
from jax.experimental.pallas import tpu as pltpu
from jax.experimental.pallas import tpu_sc as plsc

B, S, P, L = 16, 256, 8192, 16
D_TOK, D_OH, D_DEP = 256, 50, 50
D = D_TOK + D_OH + D_DEP
DP = 384
DW = DP
H = 128
PB = 256

NC, NS = 2, 16
NW = NC * NS
N_ROWS = L * P
RW = N_ROWS // NW
KCH = 128


NB = 2


def _gather_body(idx_hbm, table_hbm, out_hbm, idx_v, rows_v, semg, sems):
    wid = lax.axis_index("s") * NC + lax.axis_index("c")
    base = wid * RW
    pltpu.sync_copy(idx_hbm.at[pl.ds(base, RW)], idx_v)

    def rnd(r, carry):
        for b in range(NB):
            @pl.when(r > 0)
            def _wait_prev_scatter():
                pltpu.make_async_copy(
                    rows_v.at[b], out_hbm.at[pl.ds(base, KCH)], sems).wait()
            c = r * NB + b
            pltpu.async_copy(
                table_hbm.at[idx_v.at[pl.ds(c * KCH, KCH)]], rows_v.at[b],
                semg)
        for b in range(NB):
            c = r * NB + b
            pltpu.make_async_copy(
                table_hbm.at[idx_v.at[pl.ds(c * KCH, KCH)]], rows_v.at[b],
                semg).wait()
            pltpu.async_copy(
                rows_v.at[b], out_hbm.at[pl.ds(base + c * KCH, KCH)], sems)
        return carry

    lax.fori_loop(0, RW // (KCH * NB), rnd, 0)
    for b in range(NB):
        pltpu.make_async_copy(
            rows_v.at[b], out_hbm.at[pl.ds(base, KCH)], sems).wait()


def _sc_gather(flat_idx, table_w):
    mesh = plsc.VectorSubcoreMesh(core_axis_name="c", subcore_axis_name="s")
    return pl.kernel(
        _gather_body,
        out_type=jax.ShapeDtypeStruct((N_ROWS, DW), jnp.float32),
        mesh=mesh,
        scratch_types=[
            pltpu.VMEM((RW,), jnp.int32),
            pltpu.VMEM((NB, KCH, DW), jnp.float32),
            pltpu.SemaphoreType.DMA,
            pltpu.SemaphoreType.DMA,
        ],
    )(flat_idx, table_w)


def _scan_body(x_ref, w0_ref, u0_ref, b0_ref, w1_ref, u1_ref, b1_ref, out_ref):
    f32 = jnp.float32
    bf16 = jnp.bfloat16
    h_up = jnp.zeros((PB, H), f32)
    c_up = jnp.zeros((PB, H), f32)
    h_dn = jnp.zeros((PB, H), f32)
    c_dn = jnp.zeros((PB, H), f32)
    h_dn_first = jnp.zeros((PB, H), f32)
    b0 = b0_ref[0, :]
    b1 = b1_ref[0, :]

    def cell(x_t, h, c, w_ref, u_ref, b):
        z = jnp.dot(x_t.astype(bf16), w_ref[...], preferred_element_type=f32)
        z += jnp.dot(h.astype(bf16), u_ref[...], preferred_element_type=f32)
        z += b
        i = jax.nn.sigmoid(z[:, 0 * H:1 * H])
        o = jax.nn.sigmoid(z[:, 1 * H:2 * H])
        u = jnp.tanh(z[:, 2 * H:3 * H])
        f = jax.nn.sigmoid(z[:, 3 * H:4 * H])
        c_new = i * u + f * c
        h_new = o * jnp.tanh(c_new)
        return h_new, c_new

    for t in range(L):
        h_up, c_up = cell(x_ref[t], h_up, c_up, w0_ref, u0_ref, b0)
        h_dn, c_dn = cell(x_ref[L - 1 - t], h_dn, c_dn, w1_ref, u1_ref, b1)
        if t == 0:
            h_dn_first = h_dn

    out_ref[:, 0 * H:1 * H] = h_up
    out_ref[:, 1 * H:2 * H] = h_dn
    out_ref[:, 2 * H:3 * H] = h_dn_first


def _lstm_scan(x, w0, u0, b0, w1, u1, b1, interpret=False):
    grid = (P // PB,)
    return pl.pallas_call(
        _scan_body,
        grid=grid,
        in_specs=[
            pl.BlockSpec((L, PB, DP), lambda i: (0, i, 0)),
            pl.BlockSpec((DP, 4 * H), lambda i: (0, 0)),
            pl.BlockSpec((H, 4 * H), lambda i: (0, 0)),
            pl.BlockSpec((1, 4 * H), lambda i: (0, 0)),
            pl.BlockSpec((DP, 4 * H), lambda i: (0, 0)),
            pl.BlockSpec((H, 4 * H), lambda i: (0, 0)),
            pl.BlockSpec((1, 4 * H), lambda i: (0, 0)),
        ],
        out_specs=pl.BlockSpec((PB, 3 * H), lambda i: (i, 0)),
        out_shape=jax.ShapeDtypeStruct((P, 3 * H), jnp.float32),
        interpret=interpret,
    )(x, w0, u0, b0, w1, u1, b1)


def kernel(token_embs, dep_embs, one_hot_embs, roots, token_mask, deplinks,
           path_idx, path_batch,
           W_iou0, U_iou0, b_iou0, W_f0, U_f0, b_f0,
           W_iou1, U_iou1, b_iou1, W_f1, U_f1, b_f1):
    f32 = jnp.float32
    bf16 = jnp.bfloat16
    node = jnp.concatenate((token_embs, one_hot_embs, dep_embs), axis=-1)
    node = node * token_mask[..., None].astype(f32)
    table = jnp.pad(node.reshape(B * S, D), ((0, 0), (0, DP - D)))

    def wcat(Wiou, Wf):
        return jnp.pad(jnp.concatenate((Wiou, Wf), axis=1),
                       ((0, DP - D), (0, 0))).astype(bf16)

    w0 = wcat(W_iou0, W_f0)
    w1 = wcat(W_iou1, W_f1)
    u0 = jnp.concatenate((U_iou0, U_f0), axis=1).astype(bf16)
    u1 = jnp.concatenate((U_iou1, U_f1), axis=1).astype(bf16)
    b0 = jnp.concatenate((b_iou0, b_f0))[None, :].astype(f32)
    b1 = jnp.concatenate((b_iou1, b_f1))[None, :].astype(f32)

    flat_idx = (path_batch[None, :] * S + path_idx.T).astype(jnp.int32)
    x = _sc_gather(flat_idx.reshape(-1), table).reshape(L, P, DP)

    return _lstm_scan(x, w0, u0, b0, w1, u1, b1)


# speedup vs baseline: 7.1611x; 1.0098x over previous
import functools

import jax
import jax.numpy as jnp
from jax import lax
from jax.experimental import pallas as pl
from jax.experimental.pallas import tpu as pltpu
from jax.experimental.pallas import tpu_sc as plsc

B, S, P, L = 16, 256, 8192, 16
D_TOK, D_OH, D_DEP = 256, 50, 50
D = D_TOK + D_OH + D_DEP
DP = 384
DW = DP
H = 128
PB = 512

NC, NS = 2, 16
NW = NC * NS
N_ROWS = L * P
RW = N_ROWS // NW
KCH = 128


NB = 2


def _gather_body(idx_hbm, table_hbm, out_hbm, idx_v, rows_v, semg, sems):
    wid = lax.axis_index("s") * NC + lax.axis_index("c")
    base = wid * RW
    pltpu.sync_copy(idx_hbm.at[pl.ds(base, RW)], idx_v)

    def rnd(r, carry):
        for b in range(NB):
            @pl.when(r > 0)
            def _wait_prev_scatter():
                pltpu.make_async_copy(
                    rows_v.at[b], out_hbm.at[pl.ds(base, KCH)], sems).wait()
            c = r * NB + b
            pltpu.async_copy(
                table_hbm.at[idx_v.at[pl.ds(c * KCH, KCH)]], rows_v.at[b],
                semg)
        for b in range(NB):
            c = r * NB + b
            pltpu.make_async_copy(
                table_hbm.at[idx_v.at[pl.ds(c * KCH, KCH)]], rows_v.at[b],
                semg).wait()
            pltpu.async_copy(
                rows_v.at[b], out_hbm.at[pl.ds(base + c * KCH, KCH)], sems)
        return carry

    lax.fori_loop(0, RW // (KCH * NB), rnd, 0)
    for b in range(NB):
        pltpu.make_async_copy(
            rows_v.at[b], out_hbm.at[pl.ds(base, KCH)], sems).wait()


def _sc_gather(flat_idx, table_w):
    mesh = plsc.VectorSubcoreMesh(core_axis_name="c", subcore_axis_name="s")
    return pl.kernel(
        _gather_body,
        out_type=jax.ShapeDtypeStruct((N_ROWS, DW), jnp.float32),
        mesh=mesh,
        scratch_types=[
            pltpu.VMEM((RW,), jnp.int32),
            pltpu.VMEM((NB, KCH, DW), jnp.float32),
            pltpu.SemaphoreType.DMA,
            pltpu.SemaphoreType.DMA,
        ],
    )(flat_idx, table_w)


def _scan_body(x_ref, w0_ref, u0_ref, w1_ref, u1_ref, out_ref):
    f32 = jnp.float32
    bf16 = jnp.bfloat16
    h_up = jnp.zeros((PB, H), f32)
    c_up = jnp.zeros((PB, H), f32)
    h_dn = jnp.zeros((PB, H), f32)
    c_dn = jnp.zeros((PB, H), f32)
    h_dn_first = jnp.zeros((PB, H), f32)

    def cell(x_t, h, c, w_ref, u_ref):
        z = jnp.dot(x_t.astype(bf16), w_ref[...], preferred_element_type=f32)
        z += jnp.dot(h.astype(bf16), u_ref[...], preferred_element_type=f32)
        i = 0.5 * jnp.tanh(z[:, 0 * H:1 * H]) + 0.5
        o = 0.5 * jnp.tanh(z[:, 1 * H:2 * H]) + 0.5
        u = jnp.tanh(z[:, 2 * H:3 * H])
        f = 0.5 * jnp.tanh(z[:, 3 * H:4 * H]) + 0.5
        c_new = i * u + f * c
        h_new = o * jnp.tanh(c_new)
        return h_new, c_new

    for t in range(L):
        h_up, c_up = cell(x_ref[t], h_up, c_up, w0_ref, u0_ref)
        h_dn, c_dn = cell(x_ref[L - 1 - t], h_dn, c_dn, w1_ref, u1_ref)
        if t == 0:
            h_dn_first = h_dn

    out_ref[:, 0 * H:1 * H] = h_up
    out_ref[:, 1 * H:2 * H] = h_dn
    out_ref[:, 2 * H:3 * H] = h_dn_first


def _lstm_scan(x, w0, u0, w1, u1, interpret=False):
    grid = (P // PB,)
    return pl.pallas_call(
        _scan_body,
        grid=grid,
        in_specs=[
            pl.BlockSpec((L, PB, DP), lambda i: (0, i, 0)),
            pl.BlockSpec((DP, 4 * H), lambda i: (0, 0)),
            pl.BlockSpec((H, 4 * H), lambda i: (0, 0)),
            pl.BlockSpec((DP, 4 * H), lambda i: (0, 0)),
            pl.BlockSpec((H, 4 * H), lambda i: (0, 0)),
        ],
        out_specs=pl.BlockSpec((PB, 3 * H), lambda i: (i, 0)),
        out_shape=jax.ShapeDtypeStruct((P, 3 * H), jnp.float32),
        interpret=interpret,
    )(x, w0, u0, w1, u1)


def kernel(token_embs, dep_embs, one_hot_embs, roots, token_mask, deplinks,
           path_idx, path_batch,
           W_iou0, U_iou0, b_iou0, W_f0, U_f0, b_f0,
           W_iou1, U_iou1, b_iou1, W_f1, U_f1, b_f1):
    f32 = jnp.float32
    bf16 = jnp.bfloat16
    node = jnp.concatenate((token_embs, one_hot_embs, dep_embs), axis=-1)
    node = node * token_mask[..., None].astype(f32)
    table = jnp.pad(node.reshape(B * S, D), ((0, 0), (0, DP - D)))
    table = table.at[:, D].set(1.0)

    gate_scale = jnp.concatenate((jnp.full((2 * H,), 0.5), jnp.ones((H,)),
                                  jnp.full((H,), 0.5)))[None, :]

    def wcat(Wiou, Wf, biou, bf):
        w = jnp.pad(jnp.concatenate((Wiou, Wf), axis=1),
                    ((0, DP - D), (0, 0)))
        w = w.at[D, :].set(jnp.concatenate((biou, bf)))
        return (w * gate_scale).astype(bf16)

    w0 = wcat(W_iou0, W_f0, b_iou0, b_f0)
    w1 = wcat(W_iou1, W_f1, b_iou1, b_f1)
    u0 = (jnp.concatenate((U_iou0, U_f0), axis=1) * gate_scale).astype(bf16)
    u1 = (jnp.concatenate((U_iou1, U_f1), axis=1) * gate_scale).astype(bf16)

    flat_idx = (path_batch[None, :] * S + path_idx.T).astype(jnp.int32)
    x = _sc_gather(flat_idx.reshape(-1), table).reshape(L, P, DP)

    return _lstm_scan(x, w0, u0, w1, u1)


# speedup vs baseline: 8.2571x; 1.1531x over previous
import functools

import jax
import jax.numpy as jnp
from jax import lax
from jax.experimental import pallas as pl
from jax.experimental.pallas import tpu as pltpu
from jax.experimental.pallas import tpu_sc as plsc

B, S, P, L = 16, 256, 8192, 16
D_TOK, D_OH, D_DEP = 256, 50, 50
D = D_TOK + D_OH + D_DEP
DP = 384
H = 128
PB = 512
C = 4
PC = P // C

NC, NS = 2, 16
NW = NC * NS
KCH = 128
NB = 2


def _gather_body(rw, idx_hbm, table_hbm, out_hbm, idx_v, rows_v, semg, sems):
    wid = lax.axis_index("s") * NC + lax.axis_index("c")
    base = wid * rw
    pltpu.sync_copy(idx_hbm.at[pl.ds(base, rw)], idx_v)

    def rnd(r, carry):
        for b in range(NB):
            @pl.when(r > 0)
            def _wait_prev_scatter():
                pltpu.make_async_copy(
                    rows_v.at[b], out_hbm.at[pl.ds(base, KCH)], sems).wait()
            c = r * NB + b
            pltpu.async_copy(
                table_hbm.at[idx_v.at[pl.ds(c * KCH, KCH)]], rows_v.at[b],
                semg)
        for b in range(NB):
            c = r * NB + b
            pltpu.make_async_copy(
                table_hbm.at[idx_v.at[pl.ds(c * KCH, KCH)]], rows_v.at[b],
                semg).wait()
            pltpu.async_copy(
                rows_v.at[b], out_hbm.at[pl.ds(base + c * KCH, KCH)], sems)
        return carry

    lax.fori_loop(0, rw // (KCH * NB), rnd, 0)
    for b in range(NB):
        pltpu.make_async_copy(
            rows_v.at[b], out_hbm.at[pl.ds(base, KCH)], sems).wait()


def _sc_gather(flat_idx, table):
    n_rows = flat_idx.shape[0]
    rw = n_rows // NW
    mesh = plsc.VectorSubcoreMesh(core_axis_name="c", subcore_axis_name="s")
    return pl.kernel(
        functools.partial(_gather_body, rw),
        out_type=jax.ShapeDtypeStruct((n_rows, DP), jnp.float32),
        mesh=mesh,
        scratch_types=[
            pltpu.VMEM((rw,), jnp.int32),
            pltpu.VMEM((NB, KCH, DP), jnp.float32),
            pltpu.SemaphoreType.DMA,
            pltpu.SemaphoreType.DMA,
        ],
    )(flat_idx, table)


def _scan_body(x_ref, w0_ref, u0_ref, w1_ref, u1_ref, out_ref):
    f32 = jnp.float32
    bf16 = jnp.bfloat16
    h_up = jnp.zeros((PB, H), f32)
    c_up = jnp.zeros((PB, H), f32)
    h_dn = jnp.zeros((PB, H), f32)
    c_dn = jnp.zeros((PB, H), f32)
    h_dn_first = jnp.zeros((PB, H), f32)

    def cell(x_t, h, c, w_ref, u_ref):
        z = jnp.dot(x_t.astype(bf16), w_ref[...], preferred_element_type=f32)
        z += jnp.dot(h.astype(bf16), u_ref[...], preferred_element_type=f32)
        i = 0.5 * jnp.tanh(z[:, 0 * H:1 * H]) + 0.5
        o = 0.5 * jnp.tanh(z[:, 1 * H:2 * H]) + 0.5
        u = jnp.tanh(z[:, 2 * H:3 * H])
        f = 0.5 * jnp.tanh(z[:, 3 * H:4 * H]) + 0.5
        c_new = i * u + f * c
        h_new = o * jnp.tanh(c_new)
        return h_new, c_new

    for t in range(L):
        h_up, c_up = cell(x_ref[t], h_up, c_up, w0_ref, u0_ref)
        h_dn, c_dn = cell(x_ref[L - 1 - t], h_dn, c_dn, w1_ref, u1_ref)
        if t == 0:
            h_dn_first = h_dn

    out_ref[:, 0 * H:1 * H] = h_up
    out_ref[:, 1 * H:2 * H] = h_dn
    out_ref[:, 2 * H:3 * H] = h_dn_first


def _lstm_scan(x, w0, u0, w1, u1, interpret=False):
    grid = (PC // PB,)
    return pl.pallas_call(
        _scan_body,
        grid=grid,
        in_specs=[
            pl.BlockSpec((L, PB, DP), lambda i: (0, i, 0)),
            pl.BlockSpec((DP, 4 * H), lambda i: (0, 0)),
            pl.BlockSpec((H, 4 * H), lambda i: (0, 0)),
            pl.BlockSpec((DP, 4 * H), lambda i: (0, 0)),
            pl.BlockSpec((H, 4 * H), lambda i: (0, 0)),
        ],
        out_specs=pl.BlockSpec((PB, 3 * H), lambda i: (i, 0)),
        out_shape=jax.ShapeDtypeStruct((PC, 3 * H), jnp.float32),
        interpret=interpret,
    )(x, w0, u0, w1, u1)


def kernel(token_embs, dep_embs, one_hot_embs, roots, token_mask, deplinks,
           path_idx, path_batch,
           W_iou0, U_iou0, b_iou0, W_f0, U_f0, b_f0,
           W_iou1, U_iou1, b_iou1, W_f1, U_f1, b_f1):
    f32 = jnp.float32
    bf16 = jnp.bfloat16
    node = jnp.concatenate((token_embs, one_hot_embs, dep_embs), axis=-1)
    node = node * token_mask[..., None].astype(f32)
    table = jnp.pad(node.reshape(B * S, D), ((0, 0), (0, DP - D)))
    table = table.at[:, D].set(1.0)

    gate_scale = jnp.concatenate((jnp.full((2 * H,), 0.5), jnp.ones((H,)),
                                  jnp.full((H,), 0.5)))[None, :]

    def wcat(Wiou, Wf, biou, bf):
        w = jnp.pad(jnp.concatenate((Wiou, Wf), axis=1),
                    ((0, DP - D), (0, 0)))
        w = w.at[D, :].set(jnp.concatenate((biou, bf)))
        return (w * gate_scale).astype(bf16)

    w0 = wcat(W_iou0, W_f0, b_iou0, b_f0)
    w1 = wcat(W_iou1, W_f1, b_iou1, b_f1)
    u0 = (jnp.concatenate((U_iou0, U_f0), axis=1) * gate_scale).astype(bf16)
    u1 = (jnp.concatenate((U_iou1, U_f1), axis=1) * gate_scale).astype(bf16)

    flat_idx = (path_batch[None, :] * S + path_idx.T).astype(jnp.int32)
    outs = []
    for c in range(C):
        idx_c = flat_idx[:, c * PC:(c + 1) * PC].reshape(-1)
        x_c = _sc_gather(idx_c, table).reshape(L, PC, DP)
        outs.append(_lstm_scan(x_c, w0, u0, w1, u1))
    return jnp.concatenate(outs, axis=0)


# speedup vs baseline: 9.3139x; 1.1280x over previous
import functools

import jax
import jax.numpy as jnp
from jax import lax
from jax.experimental import pallas as pl
from jax.experimental.pallas import tpu as pltpu
from jax.experimental.pallas import tpu_sc as plsc

B, S, P, L = 16, 256, 8192, 16
D_TOK, D_OH, D_DEP = 256, 50, 50
D = D_TOK + D_OH + D_DEP
DP = 384
DPW = 256
H = 128
PB = 512
C = 4
PC = P // C

NC, NS = 2, 16
NW = NC * NS
KCH = 128
NB = 2


def _gather_body(rw, idx_hbm, table_hbm, out_hbm, idx_v, rows_v, semg, sems):
    wid = lax.axis_index("s") * NC + lax.axis_index("c")
    base = wid * rw
    pltpu.sync_copy(idx_hbm.at[pl.ds(base, rw)], idx_v)

    def rnd(r, carry):
        for b in range(NB):
            @pl.when(r > 0)
            def _wait_prev_scatter():
                pltpu.make_async_copy(
                    rows_v.at[b], out_hbm.at[pl.ds(base, KCH)], sems).wait()
            c = r * NB + b
            pltpu.async_copy(
                table_hbm.at[idx_v.at[pl.ds(c * KCH, KCH)]], rows_v.at[b],
                semg)
        for b in range(NB):
            c = r * NB + b
            pltpu.make_async_copy(
                table_hbm.at[idx_v.at[pl.ds(c * KCH, KCH)]], rows_v.at[b],
                semg).wait()
            pltpu.async_copy(
                rows_v.at[b], out_hbm.at[pl.ds(base + c * KCH, KCH)], sems)
        return carry

    lax.fori_loop(0, rw // (KCH * NB), rnd, 0)
    for b in range(NB):
        pltpu.make_async_copy(
            rows_v.at[b], out_hbm.at[pl.ds(base, KCH)], sems).wait()


def _sc_gather(flat_idx, table):
    n_rows = flat_idx.shape[0]
    rw = n_rows // NW
    mesh = plsc.VectorSubcoreMesh(core_axis_name="c", subcore_axis_name="s")
    return pl.kernel(
        functools.partial(_gather_body, rw),
        out_type=jax.ShapeDtypeStruct((n_rows, DPW), jnp.float32),
        mesh=mesh,
        scratch_types=[
            pltpu.VMEM((rw,), jnp.int32),
            pltpu.VMEM((NB, KCH, DPW), jnp.float32),
            pltpu.SemaphoreType.DMA,
            pltpu.SemaphoreType.DMA,
        ],
    )(flat_idx, table)


def _scan_body(x_ref, wa0_ref, wb0_ref, u0_ref, wa1_ref, wb1_ref, u1_ref,
               out_ref):
    f32 = jnp.float32
    bf16 = jnp.bfloat16
    HN = 2
    HPB = PB // HN

    def cell(x_t, h, c, wa_ref, wb_ref, u_ref):
        xi = lax.bitcast_convert_type(x_t, jnp.int32)
        xa = lax.bitcast_convert_type(xi << 16, f32).astype(bf16)
        xo = lax.bitcast_convert_type(xi[:, :H] & jnp.int32(-65536),
                                      f32).astype(bf16)
        z = jnp.dot(xa, wa_ref[...], preferred_element_type=f32)
        z += jnp.dot(xo, wb_ref[...], preferred_element_type=f32)
        z += jnp.dot(h.astype(bf16), u_ref[...], preferred_element_type=f32)
        i = 0.5 * jnp.tanh(z[:, 0 * H:1 * H]) + 0.5
        o = 0.5 * jnp.tanh(z[:, 1 * H:2 * H]) + 0.5
        u = jnp.tanh(z[:, 2 * H:3 * H])
        f = 0.5 * jnp.tanh(z[:, 3 * H:4 * H]) + 0.5
        c_new = i * u + f * c
        h_new = o * jnp.tanh(c_new)
        return h_new, c_new

    zero = jnp.zeros((HPB, H), f32)
    st = [[zero, zero, zero, zero, zero] for _ in range(HN)]
    for t in range(L):
        for g in range(HN):
            r = slice(g * HPB, (g + 1) * HPB)
            s = st[g]
            s[0], s[1] = cell(x_ref[t, r], s[0], s[1],
                              wa0_ref, wb0_ref, u0_ref)
            s[2], s[3] = cell(x_ref[L - 1 - t, r], s[2], s[3],
                              wa1_ref, wb1_ref, u1_ref)
            if t == 0:
                s[4] = s[2]

    for g in range(HN):
        r = slice(g * HPB, (g + 1) * HPB)
        out_ref[r, 0 * H:1 * H] = st[g][0]
        out_ref[r, 1 * H:2 * H] = st[g][2]
        out_ref[r, 2 * H:3 * H] = st[g][4]


def _lstm_scan(x, wa0, wb0, u0, wa1, wb1, u1, interpret=False):
    grid = (PC // PB,)
    wspec = lambda n: pl.BlockSpec((n, 4 * H), lambda i: (0, 0))
    return pl.pallas_call(
        _scan_body,
        grid=grid,
        in_specs=[
            pl.BlockSpec((L, PB, DPW), lambda i: (0, i, 0)),
            wspec(2 * H), wspec(H), wspec(H),
            wspec(2 * H), wspec(H), wspec(H),
        ],
        out_specs=pl.BlockSpec((PB, 3 * H), lambda i: (i, 0)),
        out_shape=jax.ShapeDtypeStruct((PC, 3 * H), jnp.float32),
        interpret=interpret,
    )(x, wa0, wb0, u0, wa1, wb1, u1)


def kernel(token_embs, dep_embs, one_hot_embs, roots, token_mask, deplinks,
           path_idx, path_batch,
           W_iou0, U_iou0, b_iou0, W_f0, U_f0, b_f0,
           W_iou1, U_iou1, b_iou1, W_f1, U_f1, b_f1):
    f32 = jnp.float32
    bf16 = jnp.bfloat16
    node = jnp.concatenate((token_embs, one_hot_embs, dep_embs), axis=-1)
    node = node * token_mask[..., None].astype(f32)
    table = jnp.pad(node.reshape(B * S, D), ((0, 0), (0, 2 * DPW - D)))
    table = table.at[:, D].set(1.0).astype(bf16)
    table_w = lax.bitcast_convert_type(
        jnp.stack((table[:, :DPW], table[:, DPW:]), axis=-1), f32)

    gate_scale = jnp.concatenate((jnp.full((2 * H,), 0.5), jnp.ones((H,)),
                                  jnp.full((H,), 0.5)))[None, :]

    def wcat(Wiou, Wf, biou, bf):
        w = jnp.pad(jnp.concatenate((Wiou, Wf), axis=1),
                    ((0, DP - D), (0, 0)))
        w = w.at[D, :].set(jnp.concatenate((biou, bf)))
        w = (w * gate_scale).astype(bf16)
        return w[:2 * H], w[2 * H:]

    wa0, wb0 = wcat(W_iou0, W_f0, b_iou0, b_f0)
    wa1, wb1 = wcat(W_iou1, W_f1, b_iou1, b_f1)
    u0 = (jnp.concatenate((U_iou0, U_f0), axis=1) * gate_scale).astype(bf16)
    u1 = (jnp.concatenate((U_iou1, U_f1), axis=1) * gate_scale).astype(bf16)

    flat_idx = (path_batch[None, :] * S + path_idx.T).astype(jnp.int32)
    outs = []
    for c in range(C):
        idx_c = flat_idx[:, c * PC:(c + 1) * PC].reshape(-1)
        x_c = _sc_gather(idx_c, table_w).reshape(L, PC, DPW)
        outs.append(_lstm_scan(x_c, wa0, wb0, u0, wa1, wb1, u1))
    return jnp.concatenate(outs, axis=0)
